# baseline (speedup 1.0000x reference)
import jax
import jax.numpy as jnp
from jax.experimental import pallas as pl
from jax.experimental.pallas import tpu as pltpu

_F = 128
_OUT = 64
_N = 10000
_C = 2000
_NC = _N // _C


def _body(x_hbm, wz_ref, wh_ref, wl_ref, bxz_ref, bhz_ref, bxh_ref, bhh_ref,
          bl_ref, out_hbm, xbuf, obuf, in_sem, out_sem):
    def copy_in(slot, idx):
        return pltpu.make_async_copy(
            x_hbm.at[pl.ds(idx * _C, _C), :], xbuf.at[slot], in_sem.at[slot])

    def copy_out(slot, idx):
        return pltpu.make_async_copy(
            obuf.at[slot], out_hbm.at[pl.ds(idx * _C, _C), :], out_sem.at[slot])

    bz = bxz_ref[0] + bhz_ref[0]
    bh = bxh_ref[0] + bhh_ref[0]
    bl = bl_ref[0]
    wz = wz_ref[:]
    wh = wh_ref[:]
    wl = wl_ref[:]

    copy_in(0, 0).start()
    for i in range(_NC):
        slot = i % 2
        if i + 1 < _NC:
            copy_in((i + 1) % 2, i + 1).start()
        copy_in(slot, i).wait()
        xb = xbuf[slot]
        az = jnp.dot(xb, wz, preferred_element_type=jnp.float32)
        ah = jnp.dot(xb, wh, preferred_element_type=jnp.float32)
        z = jax.nn.sigmoid(az + bz)
        t = jnp.tanh(ah + bh)
        h = jnp.maximum((1.0 - z) * t, 0.0)
        if i >= 2:
            copy_out(slot, i - 2).wait()
        obuf[slot] = jnp.dot(h, wl, preferred_element_type=jnp.float32) + bl
        copy_out(slot, i).start()
    copy_out((_NC - 2) % 2, _NC - 2).wait()
    copy_out((_NC - 1) % 2, _NC - 1).wait()


def kernel(x, edge_index, edge_weight, W_xz, b_xz, W_hz, b_hz, W_xr, b_xr,
           W_hr, b_hr, W_xh, b_xh, W_hh, b_hh, W_lin, b_lin):
    del edge_index, edge_weight, W_hz, W_xr, b_xr, W_hr, b_hr, W_hh

    vmem = pl.BlockSpec(memory_space=pltpu.MemorySpace.VMEM)
    out = pl.pallas_call(
        _body,
        in_specs=[
            pl.BlockSpec(memory_space=pltpu.MemorySpace.HBM),
            vmem, vmem, vmem, vmem, vmem, vmem, vmem, vmem,
        ],
        out_specs=pl.BlockSpec(memory_space=pltpu.MemorySpace.HBM),
        out_shape=jax.ShapeDtypeStruct((_N, _OUT), jnp.float32),
        scratch_shapes=[
            pltpu.VMEM((2, _C, _F), jnp.float32),
            pltpu.VMEM((2, _C, _OUT), jnp.float32),
            pltpu.SemaphoreType.DMA((2,)),
            pltpu.SemaphoreType.DMA((2,)),
        ],
    )(x, W_xz, W_xh, W_lin, b_xz.reshape(1, _F), b_hz.reshape(1, _F),
      b_xh.reshape(1, _F), b_hh.reshape(1, _F), b_lin.reshape(1, _OUT))
    return (out,)
